# baseline (speedup 1.0000x reference)
import functools

import jax
import jax.numpy as jnp
from jax.experimental import pallas as pl
from jax.experimental.pallas import tpu as pltpu
from jax.experimental.pallas import tpu_sc as plsc


def _tc_body(af_ref, gt_ref, cls_ref, tx_ref, ty_ref, tw_ref, th_ref, tcls_ref):
    ax = af_ref[0]
    ay = af_ref[1]
    aw = af_ref[2]
    ah = af_ref[3]
    ax2 = ax + aw
    ay2 = ay + ah
    area_a = aw * ah
    n = gt_ref.shape[2]

    zero = jnp.zeros_like(ax)

    def body(j, carry):
        best, bcx, bcy, bw, bh, bcls = carry
        gx = gt_ref[0, 0, j]
        gy = gt_ref[0, 1, j]
        gw = gt_ref[0, 2, j]
        gh = gt_ref[0, 3, j]
        gx2 = gx + gw
        gy2 = gy + gh
        area_g = gw * gh
        ltx = jnp.maximum(ax, gx)
        lty = jnp.maximum(ay, gy)
        rbx = jnp.minimum(ax2, gx2)
        rby = jnp.minimum(ay2, gy2)
        wi = jnp.maximum(rbx - ltx, 0.0)
        hi = jnp.maximum(rby - lty, 0.0)
        inter = wi * hi
        union = area_a + area_g - inter
        iou = inter / union
        upd = iou > best
        best = jnp.where(upd, iou, best)
        bcx = jnp.where(upd, gx + gw / 2.0, bcx)
        bcy = jnp.where(upd, gy + gh / 2.0, bcy)
        bw = jnp.where(upd, gw, bw)
        bh = jnp.where(upd, gh, bh)
        bcls = jnp.where(upd, cls_ref[0, 0, j], bcls)
        return best, bcx, bcy, bw, bh, bcls

    init = (jnp.full_like(ax, -1.0), zero, zero, zero, zero, zero)
    best, bcx, bcy, bw, bh, bcls = jax.lax.fori_loop(0, n, body, init, unroll=20)

    acx = ax + aw / 2.0
    acy = ay + ah / 2.0
    tx = ((bcx - acx) / aw) / 0.1
    ty = ((bcy - acy) / ah) / 0.1
    tw = jnp.log(bw / aw) / 0.2
    th = jnp.log(bh / ah) / 0.2
    tcls = jnp.where(best < 0.4, -1.0, jnp.where(best < 0.5, -2.0, bcls))
    nan_any = (
        jnp.isnan(tx) | jnp.isnan(ty) | jnp.isnan(tw) | jnp.isnan(th) | jnp.isnan(tcls)
    )
    tx_ref[0] = jnp.where(nan_any, -2.0, tx)
    ty_ref[0] = jnp.where(nan_any, -2.0, ty)
    tw_ref[0] = jnp.where(nan_any, -2.0, tw)
    th_ref[0] = jnp.where(nan_any, -2.0, th)
    tcls_ref[0] = jnp.where(nan_any, -2.0, tcls)


def _encode_tc_part(boxes, classes, anchors_pad):
    m_pad = anchors_pad.shape[0]
    b, n = classes.shape
    lanes = 128
    rows = m_pad // lanes
    rblk = next(r for r in (192, 96, 48, 32, 16, 8) if rows % r == 0)

    af = anchors_pad.T.reshape(4, rows, lanes)
    gt = boxes.transpose(0, 2, 1)

    out_sd = jax.ShapeDtypeStruct((b, rows, lanes), jnp.float32)
    outs = pl.pallas_call(
        _tc_body,
        grid=(b, rows // rblk),
        in_specs=[
            pl.BlockSpec((4, rblk, lanes), lambda i, j: (0, j, 0)),
            pl.BlockSpec((1, 4, n), lambda i, j: (i, 0, 0), memory_space=pltpu.SMEM),
            pl.BlockSpec((1, 1, n), lambda i, j: (i, 0, 0), memory_space=pltpu.SMEM),
        ],
        out_specs=[
            pl.BlockSpec((1, rblk, lanes), lambda i, j: (i, j, 0)) for _ in range(5)
        ],
        out_shape=[out_sd] * 5,
    )(af, gt, classes.reshape(b, 1, n))

    tx, ty, tw, th, tcls = outs
    box = jnp.stack([tx, ty, tw, th], axis=-1).reshape(b, m_pad, 4)
    return box, tcls.reshape(b, m_pad)


_SC_TILES = 32
_GRP = 32


def _sc_kernel_body(af_hbm, gt_hbm, tx_hbm, ty_hbm, tw_hbm, th_hbm, tc_hbm,
                    af_v, gt_v, o0_v, o1_v, o2_v, o3_v, o4_v, nb, n, mt):
    out_v = (o0_v, o1_v, o2_v, o3_v, o4_v)
    wid = jax.lax.axis_index("s") * 2 + jax.lax.axis_index("c")
    pltpu.sync_copy(af_hbm.at[wid], af_v)
    outs = (tx_hbm, ty_hbm, tw_hbm, th_hbm, tc_hbm)
    for bi in range(nb):
        pltpu.sync_copy(gt_hbm.at[bi], gt_v)

        def group(g, _):
            sls = [pl.ds(g * _GRP, 16), pl.ds(g * _GRP + 16, 16)]
            ax = [af_v[0, s] for s in sls]
            ay = [af_v[1, s] for s in sls]
            aw = [af_v[2, s] for s in sls]
            ah = [af_v[3, s] for s in sls]
            ax2 = [ax[i] + aw[i] for i in range(2)]
            ay2 = [ay[i] + ah[i] for i in range(2)]
            area = [aw[i] * ah[i] for i in range(2)]
            neg1 = jnp.full((16,), -1.0, jnp.float32)
            zero = jnp.zeros((16,), jnp.float32)
            init = ((neg1, zero, zero, zero, zero, zero),
                    (neg1, zero, zero, zero, zero, zero))

            npad16 = gt_v.shape[0] // 10

            def body(j, carry):
                base = j * 16
                g = [gt_v[pl.ds(base + f * npad16, 16)] for f in range(10)]
                (gxv, gyv, gx2v, gy2v, agv, gcxv, gcyv, lbwv, lbhv,
                 gclsv) = g
                new = []
                for i in range(2):
                    best, bcx, bcy, blw, blh, bcls = carry[i]
                    ltx = jnp.maximum(ax[i], gxv)
                    lty = jnp.maximum(ay[i], gyv)
                    rbx = jnp.minimum(ax2[i], gx2v)
                    rby = jnp.minimum(ay2[i], gy2v)
                    wi = jnp.maximum(rbx - ltx, 0.0)
                    hi = jnp.maximum(rby - lty, 0.0)
                    inter = wi * hi
                    union = area[i] + agv - inter
                    iou = inter / union
                    upd = iou > best
                    new.append((
                        jnp.where(upd, iou, best),
                        jnp.where(upd, gcxv, bcx),
                        jnp.where(upd, gcyv, bcy),
                        jnp.where(upd, lbwv, blw),
                        jnp.where(upd, lbhv, blh),
                        jnp.where(upd, gclsv, bcls),
                    ))
                return tuple(new)

            res = jax.lax.fori_loop(0, n, body, init, unroll=4)
            for i in range(2):
                best, bcx, bcy, blw, blh, bcls = res[i]
                sl = sls[i]
                law = af_v[4, sl]
                lah = af_v[5, sl]
                acx = ax[i] + aw[i] / 2.0
                acy = ay[i] + ah[i] / 2.0
                tx = ((bcx - acx) / aw[i]) / 0.1
                ty = ((bcy - acy) / ah[i]) / 0.1
                tw = (blw - law) / 0.2
                th = (blh - lah) / 0.2
                tcl = jnp.where(best < 0.4, -1.0,
                                jnp.where(best < 0.5, -2.0, bcls))
                nan_any = (tx != tx) | (ty != ty) | (tw != tw) | (th != th)
                out_v[0][sl] = jnp.where(nan_any, -2.0, tx)
                out_v[1][sl] = jnp.where(nan_any, -2.0, ty)
                out_v[2][sl] = jnp.where(nan_any, -2.0, tw)
                out_v[3][sl] = jnp.where(nan_any, -2.0, th)
                out_v[4][sl] = jnp.where(nan_any, -2.0, tcl)
            return 0

        jax.lax.fori_loop(0, mt // _GRP, group, 0)
        for i in range(5):
            pltpu.sync_copy(out_v[i], outs[i].at[bi, pl.ds(wid * mt, mt)])


@functools.partial(jax.jit, static_argnums=(2, 3, 4))
def _encode_sc(af_t, gt, nb, n, s_anchors):
    mt = s_anchors // _SC_TILES
    npad = gt.shape[1] // 160
    mesh = plsc.VectorSubcoreMesh(core_axis_name="c", subcore_axis_name="s")
    out_sd = jax.ShapeDtypeStruct((nb, s_anchors), jnp.float32)
    body = functools.partial(_sc_kernel_body, nb=nb, n=n, mt=mt)
    k = pl.kernel(
        body,
        mesh=mesh,
        out_type=[out_sd] * 5,
        scratch_types=[
            pltpu.VMEM((6, mt), jnp.float32),
            pltpu.VMEM((10 * npad * 16,), jnp.float32),
        ] + [pltpu.VMEM((mt,), jnp.float32)] * 5,
    )
    return k(af_t, gt)


_SC_SPLIT = 0


def kernel(images, boxes, classes, anchors):
    del images
    return _encode(boxes, classes, anchors)


@jax.jit
def _encode(boxes, classes, anchors):
    m = anchors.shape[0]
    nb, n = classes.shape
    chunk = 12288
    m_pad = ((m + chunk - 1) // chunk) * chunk
    pad = jnp.broadcast_to(
        jnp.array([0.0, 0.0, 1.0, 1.0], jnp.float32), (m_pad - m, 4)
    )
    ap = jnp.concatenate([anchors, pad], axis=0)

    s = _SC_SPLIT
    box_parts, cls_parts = [], []
    if s > 0:
        a_sc = ap[:s]
        aw = a_sc[:, 2]
        ah = a_sc[:, 3]
        af = jnp.stack(
            [a_sc[:, 0], a_sc[:, 1], aw, ah, jnp.log(aw), jnp.log(ah)], axis=0
        )
        af_t = af.reshape(6, _SC_TILES, s // _SC_TILES).transpose(1, 0, 2)
        gw = boxes[..., 2]
        gh = boxes[..., 3]
        gt = jnp.stack(
            [
                boxes[..., 0],
                boxes[..., 1],
                boxes[..., 0] + gw,
                boxes[..., 1] + gh,
                gw * gh,
                boxes[..., 0] + gw / 2.0,
                boxes[..., 1] + gh / 2.0,
                jnp.log(gw),
                jnp.log(gh),
                classes,
            ],
            axis=1,
        )
        npad = n
        gt = jnp.broadcast_to(gt[..., None], (nb, 10, npad, 16))
        gt = gt.reshape(nb, 10 * npad * 16)
        tx, ty, tw, th, tcl = _encode_sc(af_t, gt, nb, n, s)
        box_parts.append(jnp.stack([tx, ty, tw, th], axis=-1))
        cls_parts.append(tcl)
    if s < m_pad:
        box_tc, cls_tc = _encode_tc_part(boxes, classes, ap[s:])
        box_parts.append(box_tc)
        cls_parts.append(cls_tc)
    box = jnp.concatenate(box_parts, axis=1) if len(box_parts) > 1 else box_parts[0]
    cls = jnp.concatenate(cls_parts, axis=1) if len(cls_parts) > 1 else cls_parts[0]
    return box[:, :m], cls[:, :m]


# speedup vs baseline: 1.1144x; 1.1144x over previous
import functools

import jax
import jax.numpy as jnp
from jax.experimental import pallas as pl
from jax.experimental.pallas import tpu as pltpu
from jax.experimental.pallas import tpu_sc as plsc


def _tc_body(af_ref, gt_ref, cls_ref, tx_ref, ty_ref, tw_ref, th_ref, tcls_ref):
    ax = af_ref[0]
    ay = af_ref[1]
    aw = af_ref[2]
    ah = af_ref[3]
    ax2 = ax + aw
    ay2 = ay + ah
    area_a = aw * ah
    n = gt_ref.shape[2]

    zero = jnp.zeros_like(ax)

    def body(j, carry):
        best, bcx, bcy, bw, bh, bcls = carry
        gx = gt_ref[0, 0, j]
        gy = gt_ref[0, 1, j]
        gw = gt_ref[0, 2, j]
        gh = gt_ref[0, 3, j]
        gx2 = gx + gw
        gy2 = gy + gh
        area_g = gw * gh
        ltx = jnp.maximum(ax, gx)
        lty = jnp.maximum(ay, gy)
        rbx = jnp.minimum(ax2, gx2)
        rby = jnp.minimum(ay2, gy2)
        wi = jnp.maximum(rbx - ltx, 0.0)
        hi = jnp.maximum(rby - lty, 0.0)
        inter = wi * hi
        union = area_a + area_g - inter
        iou = inter / union
        upd = iou > best
        best = jnp.where(upd, iou, best)
        bcx = jnp.where(upd, gx + gw / 2.0, bcx)
        bcy = jnp.where(upd, gy + gh / 2.0, bcy)
        bw = jnp.where(upd, gw, bw)
        bh = jnp.where(upd, gh, bh)
        bcls = jnp.where(upd, cls_ref[0, 0, j], bcls)
        return best, bcx, bcy, bw, bh, bcls

    init = (jnp.full_like(ax, -1.0), zero, zero, zero, zero, zero)
    best, bcx, bcy, bw, bh, bcls = jax.lax.fori_loop(0, n, body, init, unroll=20)

    acx = ax + aw / 2.0
    acy = ay + ah / 2.0
    tx = ((bcx - acx) / aw) / 0.1
    ty = ((bcy - acy) / ah) / 0.1
    tw = jnp.log(bw / aw) / 0.2
    th = jnp.log(bh / ah) / 0.2
    tcls = jnp.where(best < 0.4, -1.0, jnp.where(best < 0.5, -2.0, bcls))
    nan_any = (
        jnp.isnan(tx) | jnp.isnan(ty) | jnp.isnan(tw) | jnp.isnan(th) | jnp.isnan(tcls)
    )
    tx_ref[0] = jnp.where(nan_any, -2.0, tx)
    ty_ref[0] = jnp.where(nan_any, -2.0, ty)
    tw_ref[0] = jnp.where(nan_any, -2.0, tw)
    th_ref[0] = jnp.where(nan_any, -2.0, th)
    tcls_ref[0] = jnp.where(nan_any, -2.0, tcls)


def _encode_tc_part(boxes, classes, anchors_pad):
    m_pad = anchors_pad.shape[0]
    b, n = classes.shape
    lanes = 128
    rows = m_pad // lanes
    rblk = next(r for r in (96, 48, 32, 16, 8) if rows % r == 0)

    af = anchors_pad.T.reshape(4, rows, lanes)
    gt = boxes.transpose(0, 2, 1)

    out_sd = jax.ShapeDtypeStruct((b, rows, lanes), jnp.float32)
    outs = pl.pallas_call(
        _tc_body,
        grid=(b, rows // rblk),
        in_specs=[
            pl.BlockSpec((4, rblk, lanes), lambda i, j: (0, j, 0)),
            pl.BlockSpec((1, 4, n), lambda i, j: (i, 0, 0), memory_space=pltpu.SMEM),
            pl.BlockSpec((1, 1, n), lambda i, j: (i, 0, 0), memory_space=pltpu.SMEM),
        ],
        out_specs=[
            pl.BlockSpec((1, rblk, lanes), lambda i, j: (i, j, 0)) for _ in range(5)
        ],
        out_shape=[out_sd] * 5,
    )(af, gt, classes.reshape(b, 1, n))

    tx, ty, tw, th, tcls = outs
    box = jnp.stack([tx, ty, tw, th], axis=-1).reshape(b, m_pad, 4)
    return box, tcls.reshape(b, m_pad)


_SC_TILES = 32
_GRP = 32


def _sc_kernel_body(af_hbm, gt_hbm, tx_hbm, ty_hbm, tw_hbm, th_hbm, tc_hbm,
                    af_v, gt_v, o0_v, o1_v, o2_v, o3_v, o4_v, nb, n, mt):
    out_v = (o0_v, o1_v, o2_v, o3_v, o4_v)
    wid = jax.lax.axis_index("s") * 2 + jax.lax.axis_index("c")
    pltpu.sync_copy(af_hbm.at[wid], af_v)
    outs = (tx_hbm, ty_hbm, tw_hbm, th_hbm, tc_hbm)
    for bi in range(nb):
        pltpu.sync_copy(gt_hbm.at[bi], gt_v)

        def group(g, _):
            sls = [pl.ds(g * _GRP, 16), pl.ds(g * _GRP + 16, 16)]
            ax = [af_v[0, s] for s in sls]
            ay = [af_v[1, s] for s in sls]
            aw = [af_v[2, s] for s in sls]
            ah = [af_v[3, s] for s in sls]
            ax2 = [ax[i] + aw[i] for i in range(2)]
            ay2 = [ay[i] + ah[i] for i in range(2)]
            area = [aw[i] * ah[i] for i in range(2)]
            neg1 = jnp.full((16,), -1.0, jnp.float32)
            zero = jnp.zeros((16,), jnp.float32)
            init = ((neg1, zero, zero, zero, zero, zero),
                    (neg1, zero, zero, zero, zero, zero))

            npad16 = gt_v.shape[0] // 10

            def body(j, carry):
                base = j * 16
                g = [gt_v[pl.ds(base + f * npad16, 16)] for f in range(10)]
                (gxv, gyv, gx2v, gy2v, agv, gcxv, gcyv, lbwv, lbhv,
                 gclsv) = g
                new = []
                for i in range(2):
                    best, bcx, bcy, blw, blh, bcls = carry[i]
                    ltx = jnp.maximum(ax[i], gxv)
                    lty = jnp.maximum(ay[i], gyv)
                    rbx = jnp.minimum(ax2[i], gx2v)
                    rby = jnp.minimum(ay2[i], gy2v)
                    wi = jnp.maximum(rbx - ltx, 0.0)
                    hi = jnp.maximum(rby - lty, 0.0)
                    inter = wi * hi
                    union = area[i] + agv - inter
                    iou = inter / union
                    upd = iou > best
                    new.append((
                        jnp.where(upd, iou, best),
                        jnp.where(upd, gcxv, bcx),
                        jnp.where(upd, gcyv, bcy),
                        jnp.where(upd, lbwv, blw),
                        jnp.where(upd, lbhv, blh),
                        jnp.where(upd, gclsv, bcls),
                    ))
                return tuple(new)

            res = jax.lax.fori_loop(0, n, body, init, unroll=4)
            for i in range(2):
                best, bcx, bcy, blw, blh, bcls = res[i]
                sl = sls[i]
                law = af_v[4, sl]
                lah = af_v[5, sl]
                acx = ax[i] + aw[i] / 2.0
                acy = ay[i] + ah[i] / 2.0
                tx = ((bcx - acx) / aw[i]) / 0.1
                ty = ((bcy - acy) / ah[i]) / 0.1
                tw = (blw - law) / 0.2
                th = (blh - lah) / 0.2
                tcl = jnp.where(best < 0.4, -1.0,
                                jnp.where(best < 0.5, -2.0, bcls))
                nan_any = (tx != tx) | (ty != ty) | (tw != tw) | (th != th)
                out_v[0][sl] = jnp.where(nan_any, -2.0, tx)
                out_v[1][sl] = jnp.where(nan_any, -2.0, ty)
                out_v[2][sl] = jnp.where(nan_any, -2.0, tw)
                out_v[3][sl] = jnp.where(nan_any, -2.0, th)
                out_v[4][sl] = jnp.where(nan_any, -2.0, tcl)
            return 0

        jax.lax.fori_loop(0, mt // _GRP, group, 0)
        for i in range(5):
            pltpu.sync_copy(out_v[i], outs[i].at[bi, pl.ds(wid * mt, mt)])


@functools.partial(jax.jit, static_argnums=(2, 3, 4))
def _encode_sc(af_t, gt, nb, n, s_anchors):
    mt = s_anchors // _SC_TILES
    npad = gt.shape[1] // 160
    mesh = plsc.VectorSubcoreMesh(core_axis_name="c", subcore_axis_name="s")
    out_sd = jax.ShapeDtypeStruct((nb, s_anchors), jnp.float32)
    body = functools.partial(_sc_kernel_body, nb=nb, n=n, mt=mt)
    k = pl.kernel(
        body,
        mesh=mesh,
        out_type=[out_sd] * 5,
        scratch_types=[
            pltpu.VMEM((6, mt), jnp.float32),
            pltpu.VMEM((10 * npad * 16,), jnp.float32),
        ] + [pltpu.VMEM((mt,), jnp.float32)] * 5,
    )
    return k(af_t, gt)


_SC_SPLIT = 12288


def kernel(images, boxes, classes, anchors):
    del images
    return _encode(boxes, classes, anchors)


@jax.jit
def _encode(boxes, classes, anchors):
    m = anchors.shape[0]
    nb, n = classes.shape
    chunk = 12288
    m_pad = ((m + chunk - 1) // chunk) * chunk
    pad = jnp.broadcast_to(
        jnp.array([0.0, 0.0, 1.0, 1.0], jnp.float32), (m_pad - m, 4)
    )
    ap = jnp.concatenate([anchors, pad], axis=0)

    s = _SC_SPLIT
    box_parts, cls_parts = [], []
    if s > 0:
        a_sc = ap[:s]
        aw = a_sc[:, 2]
        ah = a_sc[:, 3]
        af = jnp.stack(
            [a_sc[:, 0], a_sc[:, 1], aw, ah, jnp.log(aw), jnp.log(ah)], axis=0
        )
        af_t = af.reshape(6, _SC_TILES, s // _SC_TILES).transpose(1, 0, 2)
        gw = boxes[..., 2]
        gh = boxes[..., 3]
        gt = jnp.stack(
            [
                boxes[..., 0],
                boxes[..., 1],
                boxes[..., 0] + gw,
                boxes[..., 1] + gh,
                gw * gh,
                boxes[..., 0] + gw / 2.0,
                boxes[..., 1] + gh / 2.0,
                jnp.log(gw),
                jnp.log(gh),
                classes,
            ],
            axis=1,
        )
        npad = n
        gt = jnp.broadcast_to(gt[..., None], (nb, 10, npad, 16))
        gt = gt.reshape(nb, 10 * npad * 16)
        tx, ty, tw, th, tcl = _encode_sc(af_t, gt, nb, n, s)
        box_parts.append(jnp.stack([tx, ty, tw, th], axis=-1))
        cls_parts.append(tcl)
    if s < m_pad:
        box_tc, cls_tc = _encode_tc_part(boxes, classes, ap[s:])
        box_parts.append(box_tc)
        cls_parts.append(cls_tc)
    box = jnp.concatenate(box_parts, axis=1) if len(box_parts) > 1 else box_parts[0]
    cls = jnp.concatenate(cls_parts, axis=1) if len(cls_parts) > 1 else cls_parts[0]
    return box[:, :m], cls[:, :m]
